# initial kernel scaffold (unmeasured)
import jax
import jax.numpy as jnp
from jax import lax
from jax.experimental import pallas as pl
from jax.experimental.pallas import tpu as pltpu


def kernel(
    x,
):
    def body(*refs):
        pass

    out_shape = jax.ShapeDtypeStruct(..., jnp.float32)
    return pl.pallas_call(body, out_shape=out_shape)(...)


# baseline (device time: 152449 ns/iter reference)
import jax
import jax.numpy as jnp
from jax import lax
from jax.experimental import pallas as pl
from jax.experimental.pallas import tpu as pltpu

N_DEV = 32


def kernel(x):
    m_per, n = x.shape

    def body(x_ref, out_ref, send_sems, recv_sems):
        my = lax.axis_index("i")
        left = (my - 1) % N_DEV
        right = (my + 1) % N_DEV

        barrier_sem = pltpu.get_barrier_semaphore()
        for nbr in (left, right):
            pl.semaphore_signal(
                barrier_sem, inc=1,
                device_id=(nbr,), device_id_type=pl.DeviceIdType.MESH,
            )
        pl.semaphore_wait(barrier_sem, 2)

        out_ref[pl.ds(my * m_per, m_per), :] = x_ref[...]

        for h in range(N_DEV - 1):
            origin = (my - h) % N_DEV
            rdma = pltpu.make_async_remote_copy(
                src_ref=out_ref.at[pl.ds(origin * m_per, m_per), :],
                dst_ref=out_ref.at[pl.ds(origin * m_per, m_per), :],
                send_sem=send_sems.at[h],
                recv_sem=recv_sems.at[h],
                device_id=(right,),
                device_id_type=pl.DeviceIdType.MESH,
            )
            rdma.start()
            rdma.wait()

    return pl.pallas_call(
        body,
        out_shape=jax.ShapeDtypeStruct((N_DEV * m_per, n), x.dtype),
        in_specs=[pl.BlockSpec(memory_space=pltpu.VMEM)],
        out_specs=pl.BlockSpec(memory_space=pltpu.VMEM),
        scratch_shapes=[
            pltpu.SemaphoreType.DMA((N_DEV - 1,)),
            pltpu.SemaphoreType.DMA((N_DEV - 1,)),
        ],
        compiler_params=pltpu.CompilerParams(collective_id=0),
    )(x)


# device time: 90405 ns/iter; 1.6863x vs baseline; 1.6863x over previous
import jax
import jax.numpy as jnp
from jax import lax
from jax.experimental import pallas as pl
from jax.experimental.pallas import tpu as pltpu

N_DEV = 32
N_R = N_DEV // 2
N_L = N_DEV - 1 - N_R


def _mesh_order():
    coords = []
    for z in range(4):
        for yi, y in enumerate(range(4)):
            row = [(x, y, z) for x in range(2)]
            if yi % 2:
                row.reverse()
            coords.extend(row)
    return coords


def _hamiltonian_ring():
    path = []
    for zi, z in enumerate(range(4)):
        ys = list(range(4))
        if zi % 2:
            ys.reverse()
        path.extend((y, z) for y in ys)
    ring = [(1, y, z) for (y, z) in path]
    ring += [(0, y, z) for (y, z) in reversed(path)]
    return ring


_MESH_IDX = {c: i for i, c in enumerate(_mesh_order())}
_PERM = [_MESH_IDX[c] for c in _hamiltonian_ring()]
_INV = [0] * N_DEV
for _r, _m in enumerate(_PERM):
    _INV[_m] = _r


def kernel(x):
    m_per, n = x.shape

    perm = jnp.array(_PERM, dtype=jnp.int32)
    inv = jnp.array(_INV, dtype=jnp.int32)

    my = lax.axis_index("i")
    r = inv[my]
    right = perm[(r + 1) % N_DEV]
    left = perm[(r - 1) % N_DEV]
    nbrs = jnp.stack([left, right]).astype(jnp.int32)

    hr = jnp.arange(N_R, dtype=jnp.int32)
    hl = jnp.arange(N_L, dtype=jnp.int32)
    orig_r = perm[(r - hr) % N_DEV]
    orig_l = perm[(r + hl) % N_DEV]
    rcv_r = perm[(r - 1 - hr) % N_DEV]
    rcv_l = perm[(r + 1 + hl) % N_DEV]

    def body(x_ref, nbrs_ref, orig_r_ref, orig_l_ref, rcv_r_ref, rcv_l_ref,
             out_ref, send_r, recv_r, send_l, recv_l):
        me = lax.axis_index("i")
        lft = nbrs_ref[0]
        rgt = nbrs_ref[1]

        barrier_sem = pltpu.get_barrier_semaphore()
        for nbr in (lft, rgt):
            pl.semaphore_signal(
                barrier_sem, inc=1,
                device_id=(nbr,), device_id_type=pl.DeviceIdType.MESH,
            )
        pl.semaphore_wait(barrier_sem, 2)

        out_ref[pl.ds(me * m_per, m_per), :] = x_ref[...]

        def send(h, to_right):
            origin = orig_r_ref[h] if to_right else orig_l_ref[h]
            rdma = pltpu.make_async_remote_copy(
                src_ref=out_ref.at[pl.ds(origin * m_per, m_per), :],
                dst_ref=out_ref.at[pl.ds(origin * m_per, m_per), :],
                send_sem=(send_r if to_right else send_l).at[h],
                recv_sem=(recv_r if to_right else recv_l).at[h],
                device_id=((rgt if to_right else lft),),
                device_id_type=pl.DeviceIdType.MESH,
            )
            rdma.start()
            return rdma

        def wait_recv(h, from_left):
            origin = rcv_r_ref[h] if from_left else rcv_l_ref[h]
            rdma = pltpu.make_async_remote_copy(
                src_ref=out_ref.at[pl.ds(origin * m_per, m_per), :],
                dst_ref=out_ref.at[pl.ds(origin * m_per, m_per), :],
                send_sem=(send_r if from_left else send_l).at[h],
                recv_sem=(recv_r if from_left else recv_l).at[h],
                device_id=((lft if from_left else rgt),),
                device_id_type=pl.DeviceIdType.MESH,
            )
            rdma.wait_recv()

        sends = [send(0, True), send(0, False)]
        for h in range(1, N_R):
            wait_recv(h - 1, True)
            sends.append(send(h, True))
            if h < N_L:
                wait_recv(h - 1, False)
                sends.append(send(h, False))
        wait_recv(N_R - 1, True)
        wait_recv(N_L - 1, False)

        for rdma in sends:
            rdma.wait_send()

    smem = pl.BlockSpec(memory_space=pltpu.SMEM)
    return pl.pallas_call(
        body,
        out_shape=jax.ShapeDtypeStruct((N_DEV * m_per, n), x.dtype),
        in_specs=[
            pl.BlockSpec(memory_space=pltpu.VMEM),
            smem, smem, smem, smem, smem,
        ],
        out_specs=pl.BlockSpec(memory_space=pltpu.VMEM),
        scratch_shapes=[
            pltpu.SemaphoreType.DMA((N_R,)),
            pltpu.SemaphoreType.DMA((N_R,)),
            pltpu.SemaphoreType.DMA((N_L,)),
            pltpu.SemaphoreType.DMA((N_L,)),
        ],
        compiler_params=pltpu.CompilerParams(collective_id=0),
    )(x, nbrs, orig_r, orig_l, rcv_r, rcv_l)


# device time: 65421 ns/iter; 2.3303x vs baseline; 1.3819x over previous
import jax
import jax.numpy as jnp
from jax import lax
from jax.experimental import pallas as pl
from jax.experimental.pallas import tpu as pltpu

N_DEV = 32
N_R = N_DEV // 2
N_L = N_DEV - 1 - N_R
N_SEG = 4


def _mesh_order():
    coords = []
    for z in range(4):
        for yi, y in enumerate(range(4)):
            row = [(x, y, z) for x in range(2)]
            if yi % 2:
                row.reverse()
            coords.extend(row)
    return coords


def _hamiltonian_ring():
    path = []
    for zi, z in enumerate(range(4)):
        ys = list(range(4))
        if zi % 2:
            ys.reverse()
        path.extend((y, z) for y in ys)
    ring = [(1, y, z) for (y, z) in path]
    ring += [(0, y, z) for (y, z) in reversed(path)]
    return ring


_MESH_IDX = {c: i for i, c in enumerate(_mesh_order())}
_PERM = [_MESH_IDX[c] for c in _hamiltonian_ring()]
_INV = [0] * N_DEV
for _r, _m in enumerate(_PERM):
    _INV[_m] = _r


def kernel(x):
    m_per, n = x.shape

    perm = jnp.array(_PERM, dtype=jnp.int32)
    inv = jnp.array(_INV, dtype=jnp.int32)

    my = lax.axis_index("i")
    r = inv[my]
    right = perm[(r + 1) % N_DEV]
    left = perm[(r - 1) % N_DEV]
    nbrs = jnp.stack([left, right]).astype(jnp.int32)

    hr = jnp.arange(N_R, dtype=jnp.int32)
    hl = jnp.arange(N_L, dtype=jnp.int32)
    orig_r = perm[(r - hr) % N_DEV]
    orig_l = perm[(r + hl) % N_DEV]
    rcv_r = perm[(r - 1 - hr) % N_DEV]
    rcv_l = perm[(r + 1 + hl) % N_DEV]

    def body(x_ref, nbrs_ref, orig_r_ref, orig_l_ref, rcv_r_ref, rcv_l_ref,
             out_ref, send_r, recv_r, send_l, recv_l):
        me = lax.axis_index("i")
        lft = nbrs_ref[0]
        rgt = nbrs_ref[1]

        barrier_sem = pltpu.get_barrier_semaphore()
        for nbr in (lft, rgt):
            pl.semaphore_signal(
                barrier_sem, inc=1,
                device_id=(nbr,), device_id_type=pl.DeviceIdType.MESH,
            )
        pl.semaphore_wait(barrier_sem, 2)

        out_ref[pl.ds(me * m_per, m_per), :] = x_ref[...]

        seg_rows = m_per // N_SEG

        def send(h, s, to_right):
            origin = orig_r_ref[h] if to_right else orig_l_ref[h]
            off = origin * m_per + s * seg_rows
            rdma = pltpu.make_async_remote_copy(
                src_ref=out_ref.at[pl.ds(off, seg_rows), :],
                dst_ref=out_ref.at[pl.ds(off, seg_rows), :],
                send_sem=(send_r if to_right else send_l).at[h, s],
                recv_sem=(recv_r if to_right else recv_l).at[h, s],
                device_id=((rgt if to_right else lft),),
                device_id_type=pl.DeviceIdType.MESH,
            )
            rdma.start()
            return rdma

        def wait_recv(h, s, from_left):
            origin = rcv_r_ref[h] if from_left else rcv_l_ref[h]
            off = origin * m_per + s * seg_rows
            rdma = pltpu.make_async_remote_copy(
                src_ref=out_ref.at[pl.ds(off, seg_rows), :],
                dst_ref=out_ref.at[pl.ds(off, seg_rows), :],
                send_sem=(send_r if from_left else send_l).at[h, s],
                recv_sem=(recv_r if from_left else recv_l).at[h, s],
                device_id=((lft if from_left else rgt),),
                device_id_type=pl.DeviceIdType.MESH,
            )
            rdma.wait_recv()

        sends = []
        for s in range(N_SEG):
            sends.append(send(0, s, True))
            sends.append(send(0, s, False))
        for h in range(1, N_R):
            for s in range(N_SEG):
                wait_recv(h - 1, s, True)
                sends.append(send(h, s, True))
                if h < N_L:
                    wait_recv(h - 1, s, False)
                    sends.append(send(h, s, False))
        for s in range(N_SEG):
            wait_recv(N_R - 1, s, True)
        for s in range(N_SEG):
            wait_recv(N_L - 1, s, False)

        for rdma in sends:
            rdma.wait_send()

    smem = pl.BlockSpec(memory_space=pltpu.SMEM)
    return pl.pallas_call(
        body,
        out_shape=jax.ShapeDtypeStruct((N_DEV * m_per, n), x.dtype),
        in_specs=[
            pl.BlockSpec(memory_space=pltpu.VMEM),
            smem, smem, smem, smem, smem,
        ],
        out_specs=pl.BlockSpec(memory_space=pltpu.VMEM),
        scratch_shapes=[
            pltpu.SemaphoreType.DMA((N_R, N_SEG)),
            pltpu.SemaphoreType.DMA((N_R, N_SEG)),
            pltpu.SemaphoreType.DMA((N_L, N_SEG)),
            pltpu.SemaphoreType.DMA((N_L, N_SEG)),
        ],
        compiler_params=pltpu.CompilerParams(collective_id=0),
    )(x, nbrs, orig_r, orig_l, rcv_r, rcv_l)
